# Initial kernel scaffold; baseline (speedup 1.0000x reference)
import jax
import jax.numpy as jnp
from jax.experimental import pallas as pl


def kernel(x, edge_index, W_init, b_init, W0, b0, W1, b1, W2, b2):
    raise NotImplementedError("write your pallas kernel here")


# speedup vs baseline: 5.0740x; 5.0740x over previous
import functools

import jax
import jax.numpy as jnp
from jax import lax
from jax.experimental import pallas as pl
from jax.experimental.pallas import tpu as pltpu
from jax.experimental.pallas import tpu_sc as plsc

_N = 10000
_E = 320000
_D = 128
_NC = 2
_NS = 16
_NW = _NC * _NS
_LPR = 128
_RPW = 80
_RTOT = _NW * _RPW
_EPAD = _RTOT * _LPR
_NPAD = 10240
_ZCHUNK = 64
_ZROWS = _NPAD // _NS
_KGRP = 8


def _mesh():
    return plsc.VectorSubcoreMesh(
        core_axis_name="c", subcore_axis_name="s",
        num_cores=_NC, num_subcores=_NS)


def _fill_rows(buf, value):
    rows, cols = buf.shape
    v = jnp.full((16,), value, jnp.float32)

    @pl.loop(0, rows)
    def _(i):
        @pl.loop(0, cols // 16)
        def _(j):
            buf[i, pl.ds(j * 16, 16)] = v


def _sc_segment_sum(g, src2d, dst2d):
    @functools.partial(
        pl.kernel,
        out_type=jax.ShapeDtypeStruct((_NC, _NPAD, _D), jnp.float32),
        mesh=_mesh(),
        scratch_types=[
            pltpu.VMEM((_KGRP, _LPR), jnp.int32),
            pltpu.VMEM((_KGRP, _LPR), jnp.int32),
            pltpu.VMEM((_LPR, _D), jnp.float32),
            pltpu.VMEM((_ZCHUNK, _D), jnp.float32),
            pltpu.VMEM_SHARED((_NPAD, _D), jnp.float32),
            pltpu.SemaphoreType.DMA,
        ],
    )
    def k(g_hbm, src_hbm, dst_hbm, out_hbm, isrc, idst, rows, zbuf, acc, sem):
        cid = lax.axis_index("c")
        sid = lax.axis_index("s")

        _fill_rows(zbuf, 0.0)

        @pl.loop(0, _ZROWS // _ZCHUNK)
        def _(t):
            pltpu.sync_copy(zbuf, acc.at[pl.ds(sid * _ZROWS + t * _ZCHUNK, _ZCHUNK)])

        plsc.subcore_barrier()

        base_row = cid * (_RTOT // _NC) + sid * _RPW

        @pl.loop(0, _RPW // _KGRP)
        def _(grp):
            r0 = base_row + grp * _KGRP
            pltpu.sync_copy(src_hbm.at[pl.ds(r0, _KGRP)], isrc)
            pltpu.sync_copy(dst_hbm.at[pl.ds(r0, _KGRP)], idst)
            for j in range(_KGRP):
                pltpu.async_copy(g_hbm.at[isrc.at[j]], rows, sem).wait()
                pltpu.sync_copy(rows, acc.at[idst.at[j]], add=True)

        plsc.subcore_barrier()

        pltpu.sync_copy(acc.at[pl.ds(sid * _ZROWS, _ZROWS)],
                        out_hbm.at[cid].at[pl.ds(sid * _ZROWS, _ZROWS)])

    return k(g, src2d, dst2d)


def _tc_head(xp, w_init_p, b_init, d0, d1, w0):
    def body(x_ref, wi_ref, bi_ref, d0_ref, d1_ref, w0_ref,
             ih_ref, g0_ref, dinv_ref):
        deg = d0_ref[...] + d1_ref[...] + 1.0
        dinv = lax.rsqrt(deg)
        dinv_ref[...] = dinv
        ih = jnp.dot(x_ref[...], wi_ref[...],
                     preferred_element_type=jnp.float32) + bi_ref[...]
        ih_ref[...] = ih
        y = jnp.dot(ih, w0_ref[...], preferred_element_type=jnp.float32)
        g0_ref[...] = y * dinv

    return pl.pallas_call(
        body,
        out_shape=[
            jax.ShapeDtypeStruct((_N, _D), jnp.float32),
            jax.ShapeDtypeStruct((_N, _D), jnp.float32),
            jax.ShapeDtypeStruct((_N, 1), jnp.float32),
        ],
    )(xp, w_init_p, b_init, d0, d1, w0)


def _tc_mid(p0, p1, g_prev, dinv, b, w_next):
    def body(p0_ref, p1_ref, g_ref, dinv_ref, b_ref, w_ref, out_ref):
        dinv = dinv_ref[...]
        z = dinv * (p0_ref[...] + p1_ref[...] + g_ref[...]) + b_ref[...]
        h = jnp.maximum(z, 0.0)
        out_ref[...] = dinv * jnp.dot(h, w_ref[...],
                                      preferred_element_type=jnp.float32)

    return pl.pallas_call(
        body,
        out_shape=jax.ShapeDtypeStruct((_N, _D), jnp.float32),
    )(p0, p1, g_prev, dinv, b, w_next)


def _tc_tail(p0, p1, g_prev, dinv, b, init_h):
    def body(p0_ref, p1_ref, g_ref, dinv_ref, b_ref, ih_ref, out_ref):
        z = dinv_ref[...] * (p0_ref[...] + p1_ref[...] + g_ref[...]) + b_ref[...]
        out_ref[...] = z + ih_ref[...]

    return pl.pallas_call(
        body,
        out_shape=jax.ShapeDtypeStruct((_N, _D), jnp.float32),
    )(p0, p1, g_prev, dinv, b, init_h)


def kernel(x, edge_index, W_init, b_init, W0, b0, W1, b1, W2, b2):
    src = edge_index[0].astype(jnp.int32)
    dst = edge_index[1].astype(jnp.int32)
    pad = _EPAD - _E
    src2d = jnp.concatenate([src, jnp.zeros((pad,), jnp.int32)]).reshape(_RTOT, _LPR)
    dst2d = jnp.concatenate([dst, jnp.full((pad,), _N, jnp.int32)]).reshape(_RTOT, _LPR)

    xp = jnp.pad(x, ((0, 0), (0, 8 - x.shape[1])))
    w_init_p = jnp.pad(W_init, ((0, 8 - W_init.shape[0]), (0, 0)))
    bi = b_init.reshape(1, _D)
    b0r = b0.reshape(1, _D)
    b1r = b1.reshape(1, _D)
    b2r = b2.reshape(1, _D)

    dparts = _sc_segment_sum(jnp.ones((_N, _D), jnp.float32), src2d, dst2d)
    d0 = dparts[0, :_N, :1]
    d1 = dparts[1, :_N, :1]

    init_h, g0, dinv = _tc_head(xp, w_init_p, bi, d0, d1, W0)

    p = _sc_segment_sum(g0, src2d, dst2d)
    g1 = _tc_mid(p[0, :_N], p[1, :_N], g0, dinv, b0r, W1)

    p = _sc_segment_sum(g1, src2d, dst2d)
    g2 = _tc_mid(p[0, :_N], p[1, :_N], g1, dinv, b1r, W2)

    p = _sc_segment_sum(g2, src2d, dst2d)
    out = _tc_tail(p[0, :_N], p[1, :_N], g2, dinv, b2r, init_h)

    return (out, init_h)


# speedup vs baseline: 5.6645x; 1.1164x over previous
import functools

import jax
import jax.numpy as jnp
from jax import lax
from jax.experimental import pallas as pl
from jax.experimental.pallas import tpu as pltpu
from jax.experimental.pallas import tpu_sc as plsc

_N = 10000
_E = 320000
_D = 128
_NC = 2
_NS = 16
_NW = _NC * _NS
_LPR = 128
_RPW = 80
_RTOT = _NW * _RPW
_EPAD = _RTOT * _LPR
_NPAD = 10240
_ZCHUNK = 16
_ZROWS = _NPAD // _NS
_KGRP = 16


def _mesh():
    return plsc.VectorSubcoreMesh(
        core_axis_name="c", subcore_axis_name="s",
        num_cores=_NC, num_subcores=_NS)


def _fill_rows(buf, value):
    rows, cols = buf.shape
    v = jnp.full((16,), value, jnp.float32)

    @pl.loop(0, rows)
    def _(i):
        @pl.loop(0, cols // 16)
        def _(j):
            buf[i, pl.ds(j * 16, 16)] = v


def _zero_acc(zbuf, acc, sid):
    _fill_rows(zbuf, 0.0)

    @pl.loop(0, _ZROWS // _ZCHUNK)
    def _(t):
        pltpu.sync_copy(zbuf, acc.at[pl.ds(sid * _ZROWS + t * _ZCHUNK, _ZCHUNK)])


def _drain_acc(acc, out_hbm, cid, sid):
    pltpu.sync_copy(acc.at[pl.ds(sid * _ZROWS, _ZROWS)],
                    out_hbm.at[cid].at[pl.ds(sid * _ZROWS, _ZROWS)])


def _sc_segment_sum(g, src2d, dst2d):
    @functools.partial(
        pl.kernel,
        out_type=jax.ShapeDtypeStruct((_NC, _NPAD, _D), jnp.float32),
        mesh=_mesh(),
        scratch_types=[
            pltpu.VMEM((_KGRP, _LPR), jnp.int32),
            pltpu.VMEM((_KGRP, _LPR), jnp.int32),
            [pltpu.VMEM((_LPR, _D), jnp.float32)] * 2,
            pltpu.VMEM((16, _D), jnp.float32),
            pltpu.VMEM_SHARED((_NPAD, _D), jnp.float32),
            [pltpu.SemaphoreType.DMA] * 2,
            [pltpu.SemaphoreType.DMA] * 2,
        ],
    )
    def k(g_hbm, src_hbm, dst_hbm, out_hbm, isrc, idst, bufs, zbuf, acc,
          gsem, ssem):
        cid = lax.axis_index("c")
        sid = lax.axis_index("s")

        _zero_acc(zbuf, acc, sid)
        plsc.subcore_barrier()

        base_row = cid * (_RTOT // _NC) + sid * _RPW

        @pl.loop(0, _RPW // _KGRP)
        def _(grp):
            r0 = base_row + grp * _KGRP
            pltpu.sync_copy(src_hbm.at[pl.ds(r0, _KGRP)], isrc)
            pltpu.sync_copy(dst_hbm.at[pl.ds(r0, _KGRP)], idst)

            for j in range(2):
                pltpu.async_copy(g_hbm.at[isrc.at[j]], bufs[j], gsem[j])

            for j in range(_KGRP):
                b = j % 2
                pltpu.make_async_copy(g_hbm.at[isrc.at[j]], bufs[b],
                                      gsem[b]).wait()
                pltpu.async_copy(bufs[b], acc.at[idst.at[j]], ssem[b],
                                 add=True)
                if j + 2 < _KGRP:
                    pltpu.make_async_copy(bufs[b], acc.at[idst.at[j]],
                                          ssem[b]).wait()
                    pltpu.async_copy(g_hbm.at[isrc.at[j + 2]], bufs[b],
                                     gsem[b])

            for j in (_KGRP - 2, _KGRP - 1):
                pltpu.make_async_copy(bufs[j % 2], acc.at[idst.at[j]],
                                      ssem[j % 2]).wait()

        plsc.subcore_barrier()
        _drain_acc(acc, out_hbm, cid, sid)

    return k(g, src2d, dst2d)


def _tc_head(xp, w_init_p, b_init, d0, d1, w0):
    def body(x_ref, wi_ref, bi_ref, d0_ref, d1_ref, w0_ref,
             ih_ref, g0_ref, dinv_ref):
        deg = d0_ref[...] + d1_ref[...] + 1.0
        dinv = lax.rsqrt(deg)
        dinv_ref[...] = dinv
        ih = jnp.dot(x_ref[...], wi_ref[...],
                     preferred_element_type=jnp.float32) + bi_ref[...]
        ih_ref[...] = ih
        y = jnp.dot(ih, w0_ref[...], preferred_element_type=jnp.float32)
        g0_ref[...] = y * dinv

    return pl.pallas_call(
        body,
        out_shape=[
            jax.ShapeDtypeStruct((_N, _D), jnp.float32),
            jax.ShapeDtypeStruct((_N, _D), jnp.float32),
            jax.ShapeDtypeStruct((_N, 1), jnp.float32),
        ],
    )(xp, w_init_p, b_init, d0, d1, w0)


def _tc_mid(p0, p1, g_prev, dinv, b, w_next):
    def body(p0_ref, p1_ref, g_ref, dinv_ref, b_ref, w_ref, out_ref):
        dinv = dinv_ref[...]
        z = dinv * (p0_ref[...] + p1_ref[...] + g_ref[...]) + b_ref[...]
        h = jnp.maximum(z, 0.0)
        out_ref[...] = dinv * jnp.dot(h, w_ref[...],
                                      preferred_element_type=jnp.float32)

    return pl.pallas_call(
        body,
        out_shape=jax.ShapeDtypeStruct((_N, _D), jnp.float32),
    )(p0, p1, g_prev, dinv, b, w_next)


def _tc_tail(p0, p1, g_prev, dinv, b, init_h):
    def body(p0_ref, p1_ref, g_ref, dinv_ref, b_ref, ih_ref, out_ref):
        z = dinv_ref[...] * (p0_ref[...] + p1_ref[...] + g_ref[...]) + b_ref[...]
        out_ref[...] = z + ih_ref[...]

    return pl.pallas_call(
        body,
        out_shape=jax.ShapeDtypeStruct((_N, _D), jnp.float32),
    )(p0, p1, g_prev, dinv, b, init_h)


def kernel(x, edge_index, W_init, b_init, W0, b0, W1, b1, W2, b2):
    src = edge_index[0].astype(jnp.int32)
    dst = edge_index[1].astype(jnp.int32)
    pad = _EPAD - _E
    src2d = jnp.concatenate([src, jnp.zeros((pad,), jnp.int32)]).reshape(_RTOT, _LPR)
    dst2d = jnp.concatenate([dst, jnp.full((pad,), _N, jnp.int32)]).reshape(_RTOT, _LPR)

    xp = jnp.pad(x, ((0, 0), (0, 8 - x.shape[1])))
    w_init_p = jnp.pad(W_init, ((0, 8 - W_init.shape[0]), (0, 0)))
    bi = b_init.reshape(1, _D)
    b0r = b0.reshape(1, _D)
    b1r = b1.reshape(1, _D)
    b2r = b2.reshape(1, _D)

    dparts = _sc_segment_sum(jnp.ones((_N, _D), jnp.float32), src2d, dst2d)
    d0 = dparts[0, :_N, :1]
    d1 = dparts[1, :_N, :1]

    init_h, g0, dinv = _tc_head(xp, w_init_p, bi, d0, d1, W0)

    p = _sc_segment_sum(g0, src2d, dst2d)
    g1 = _tc_mid(p[0, :_N], p[1, :_N], g0, dinv, b0r, W1)

    p = _sc_segment_sum(g1, src2d, dst2d)
    g2 = _tc_mid(p[0, :_N], p[1, :_N], g1, dinv, b1r, W2)

    p = _sc_segment_sum(g2, src2d, dst2d)
    out = _tc_tail(p[0, :_N], p[1, :_N], g2, dinv, b2r, init_h)

    return (out, init_h)


# speedup vs baseline: 7.4979x; 1.3237x over previous
import dataclasses
import functools

import jax
import jax.numpy as jnp
from jax import lax
from jax.experimental import pallas as pl
from jax.experimental.pallas import tpu as pltpu
from jax.experimental.pallas import tpu_sc as plsc

_N = 10000
_E = 320000
_D = 128
_NC = 2
_NS = 16
_NW = _NC * _NS
_LPR = 128
_RPW = 80
_RTOT = _NW * _RPW
_EPAD = _RTOT * _LPR
_NPAD = 10240
_ZCHUNK = 16
_ZROWS = _NPAD // _NS
_KGRP = 16


def _mesh():
    return plsc.VectorSubcoreMesh(
        core_axis_name="c", subcore_axis_name="s",
        num_cores=_NC, num_subcores=_NS)


def _sc_params():
    cp = pltpu.CompilerParams()
    if "needs_layout_passes" in pltpu.CompilerParams.__dataclass_fields__:
        cp = dataclasses.replace(cp, needs_layout_passes=False)
    return cp


def _fill_rows(buf, value):
    rows, cols = buf.shape
    v = jnp.full((16,), value, jnp.float32)

    @pl.loop(0, rows)
    def _(i):
        @pl.loop(0, cols // 16)
        def _(j):
            buf[i, pl.ds(j * 16, 16)] = v


def _zero_acc(zbuf, acc, sid):
    _fill_rows(zbuf, 0.0)

    @pl.loop(0, _ZROWS // _ZCHUNK)
    def _(t):
        pltpu.sync_copy(zbuf, acc.at[pl.ds(sid * _ZROWS + t * _ZCHUNK, _ZCHUNK)])


def _drain_acc(acc, out_hbm, cid, sid):
    pltpu.sync_copy(acc.at[pl.ds(sid * _ZROWS, _ZROWS)],
                    out_hbm.at[cid].at[pl.ds(sid * _ZROWS, _ZROWS)])


def _sc_segment_sum(g, src2d, dst2d, ones_mode):
    @functools.partial(
        pl.kernel,
        out_type=jax.ShapeDtypeStruct((_NC, _NPAD, _D), jnp.float32),
        mesh=_mesh(),
        compiler_params=_sc_params(),
        scratch_types=[
            pltpu.VMEM((_KGRP, _LPR), jnp.int32),
            pltpu.VMEM((_KGRP, _LPR), jnp.int32),
            pltpu.VMEM((8, _LPR), jnp.int32),
            [pltpu.VMEM((_LPR, _D), jnp.float32)] * 2,
            pltpu.VMEM((16, _D), jnp.float32),
            pltpu.VMEM_SHARED((_NPAD, _D), jnp.float32),
            [pltpu.SemaphoreType.DMA] * 2,
            [pltpu.SemaphoreType.DMA] * 2,
        ],
    )
    def k(g_hbm, src_hbm, dst_hbm, mode_hbm, out_hbm, isrc, idst, modebuf,
          bufs, zbuf, acc, gsem, ssem):
        cid = lax.axis_index("c")
        sid = lax.axis_index("s")

        pltpu.sync_copy(mode_hbm, modebuf)
        ones = jnp.max(modebuf[0, pl.ds(0, 16)]) > 0
        gather = jnp.logical_not(ones)

        _zero_acc(zbuf, acc, sid)

        @pl.when(ones)
        def _():
            _fill_rows(bufs[0], 1.0)
            _fill_rows(bufs[1], 1.0)

        plsc.subcore_barrier()

        base_row = cid * (_RTOT // _NC) + sid * _RPW

        @pl.loop(0, _RPW // _KGRP)
        def _(grp):
            r0 = base_row + grp * _KGRP
            pltpu.sync_copy(dst_hbm.at[pl.ds(r0, _KGRP)], idst)

            @pl.when(gather)
            def _():
                pltpu.sync_copy(src_hbm.at[pl.ds(r0, _KGRP)], isrc)
                for j in range(2):
                    pltpu.async_copy(g_hbm.at[isrc.at[j]], bufs[j], gsem[j])

            for j in range(_KGRP):
                b = j % 2

                @pl.when(gather)
                def _():
                    pltpu.make_async_copy(g_hbm.at[isrc.at[j]], bufs[b],
                                          gsem[b]).wait()

                pltpu.async_copy(bufs[b], acc.at[idst.at[j]], ssem[b],
                                 add=True)
                if j + 2 < _KGRP:
                    pltpu.make_async_copy(bufs[b], acc.at[idst.at[j]],
                                          ssem[b]).wait()

                    @pl.when(gather)
                    def _():
                        pltpu.async_copy(g_hbm.at[isrc.at[j + 2]], bufs[b],
                                         gsem[b])

            for j in (_KGRP - 2, _KGRP - 1):
                pltpu.make_async_copy(bufs[j % 2], acc.at[idst.at[j]],
                                      ssem[j % 2]).wait()

        plsc.subcore_barrier()
        _drain_acc(acc, out_hbm, cid, sid)

    return k(g, src2d, dst2d, ones_mode)


def _tc_head(xp, w_init_p, b_init, d0, d1, w0):
    def body(x_ref, wi_ref, bi_ref, d0_ref, d1_ref, w0_ref,
             ih_ref, g0_ref, dinv_ref):
        deg = d0_ref[...] + d1_ref[...] + 1.0
        dinv = lax.rsqrt(deg)
        dinv_ref[...] = dinv
        ih = jnp.dot(x_ref[...], wi_ref[...],
                     preferred_element_type=jnp.float32) + bi_ref[...]
        ih_ref[...] = ih
        y = jnp.dot(ih, w0_ref[...], preferred_element_type=jnp.float32)
        g0_ref[...] = y * dinv

    return pl.pallas_call(
        body,
        out_shape=[
            jax.ShapeDtypeStruct((_N, _D), jnp.float32),
            jax.ShapeDtypeStruct((_N, _D), jnp.float32),
            jax.ShapeDtypeStruct((_N, 1), jnp.float32),
        ],
    )(xp, w_init_p, b_init, d0, d1, w0)


def _tc_mid(p0, p1, g_prev, dinv, b, w_next):
    def body(p0_ref, p1_ref, g_ref, dinv_ref, b_ref, w_ref, out_ref):
        dinv = dinv_ref[...]
        z = dinv * (p0_ref[...] + p1_ref[...] + g_ref[...]) + b_ref[...]
        h = jnp.maximum(z, 0.0)
        out_ref[...] = dinv * jnp.dot(h, w_ref[...],
                                      preferred_element_type=jnp.float32)

    return pl.pallas_call(
        body,
        out_shape=jax.ShapeDtypeStruct((_N, _D), jnp.float32),
    )(p0, p1, g_prev, dinv, b, w_next)


def _tc_tail(p0, p1, g_prev, dinv, b, init_h):
    def body(p0_ref, p1_ref, g_ref, dinv_ref, b_ref, ih_ref, out_ref):
        z = dinv_ref[...] * (p0_ref[...] + p1_ref[...] + g_ref[...]) + b_ref[...]
        out_ref[...] = z + ih_ref[...]

    return pl.pallas_call(
        body,
        out_shape=jax.ShapeDtypeStruct((_N, _D), jnp.float32),
    )(p0, p1, g_prev, dinv, b, init_h)


def kernel(x, edge_index, W_init, b_init, W0, b0, W1, b1, W2, b2):
    src = edge_index[0].astype(jnp.int32)
    dst = edge_index[1].astype(jnp.int32)
    pad = _EPAD - _E
    src2d = jnp.concatenate([src, jnp.zeros((pad,), jnp.int32)]).reshape(_RTOT, _LPR)
    dst2d = jnp.concatenate([dst, jnp.full((pad,), _N, jnp.int32)]).reshape(_RTOT, _LPR)

    xp = jnp.pad(x, ((0, 0), (0, 8 - x.shape[1])))
    w_init_p = jnp.pad(W_init, ((0, 8 - W_init.shape[0]), (0, 0)))
    bi = b_init.reshape(1, _D)
    b0r = b0.reshape(1, _D)
    b1r = b1.reshape(1, _D)
    b2r = b2.reshape(1, _D)

    mode_hist = jnp.ones((8, _LPR), jnp.int32)
    mode_edge = jnp.zeros((8, _LPR), jnp.int32)

    dparts = _sc_segment_sum(jnp.zeros((_N, _D), jnp.float32), src2d, dst2d,
                             mode_hist)
    d0 = dparts[0, :_N, :1]
    d1 = dparts[1, :_N, :1]

    init_h, g0, dinv = _tc_head(xp, w_init_p, bi, d0, d1, W0)

    p = _sc_segment_sum(g0, src2d, dst2d, mode_edge)
    g1 = _tc_mid(p[0, :_N], p[1, :_N], g0, dinv, b0r, W1)

    p = _sc_segment_sum(g1, src2d, dst2d, mode_edge)
    g2 = _tc_mid(p[0, :_N], p[1, :_N], g1, dinv, b1r, W2)

    p = _sc_segment_sum(g2, src2d, dst2d, mode_edge)
    out = _tc_tail(p[0, :_N], p[1, :_N], g2, dinv, b2r, init_h)

    return (out, init_h)


# speedup vs baseline: 7.5442x; 1.0062x over previous
import dataclasses
import functools

import jax
import jax.numpy as jnp
from jax import lax
from jax.experimental import pallas as pl
from jax.experimental.pallas import tpu as pltpu
from jax.experimental.pallas import tpu_sc as plsc

_N = 10000
_E = 320000
_D = 128
_NC = 2
_NS = 16
_NW = _NC * _NS
_LPR = 128
_RPW = 80
_RTOT = _NW * _RPW
_EPAD = _RTOT * _LPR
_NPAD = 10240
_ZCHUNK = 64
_ZROWS = _NPAD // _NS
_KGRP = 16


def _mesh():
    return plsc.VectorSubcoreMesh(
        core_axis_name="c", subcore_axis_name="s",
        num_cores=_NC, num_subcores=_NS)


def _sc_params():
    cp = pltpu.CompilerParams()
    if "needs_layout_passes" in pltpu.CompilerParams.__dataclass_fields__:
        cp = dataclasses.replace(cp, needs_layout_passes=False)
    return cp


def _fill_rows(buf, value):
    rows, cols = buf.shape
    v = jnp.full((16,), value, jnp.float32)

    @pl.loop(0, rows)
    def _(i):
        @pl.loop(0, cols // 16)
        def _(j):
            buf[i, pl.ds(j * 16, 16)] = v


def _zero_acc(zbuf, acc, sid):
    _fill_rows(zbuf, 0.0)

    @pl.loop(0, _ZROWS // _ZCHUNK)
    def _(t):
        pltpu.sync_copy(zbuf, acc.at[pl.ds(sid * _ZROWS + t * _ZCHUNK, _ZCHUNK)])


def _drain_acc(acc, out_hbm, cid, sid):
    pltpu.sync_copy(acc.at[pl.ds(sid * _ZROWS, _ZROWS)],
                    out_hbm.at[cid].at[pl.ds(sid * _ZROWS, _ZROWS)])


def _sc_segment_sum(g, src2d, dst2d, ones_mode):
    @functools.partial(
        pl.kernel,
        out_type=jax.ShapeDtypeStruct((_NC, _NPAD, _D), jnp.float32),
        mesh=_mesh(),
        compiler_params=_sc_params(),
        scratch_types=[
            pltpu.VMEM((_KGRP, _LPR), jnp.int32),
            pltpu.VMEM((_KGRP, _LPR), jnp.int32),
            pltpu.VMEM((8, _LPR), jnp.int32),
            [pltpu.VMEM((_LPR, _D), jnp.float32)] * 2,
            pltpu.VMEM((_ZCHUNK, _D), jnp.float32),
            pltpu.VMEM_SHARED((_NPAD, _D), jnp.float32),
            [pltpu.SemaphoreType.DMA] * 2,
            [pltpu.SemaphoreType.DMA] * 2,
        ],
    )
    def k(g_hbm, src_hbm, dst_hbm, mode_hbm, out_hbm, isrc, idst, modebuf,
          bufs, zbuf, acc, gsem, ssem):
        cid = lax.axis_index("c")
        sid = lax.axis_index("s")

        pltpu.sync_copy(mode_hbm, modebuf)
        ones = jnp.max(modebuf[0, pl.ds(0, 16)]) > 0
        gather = jnp.logical_not(ones)

        base_row = cid * (_RTOT // _NC) + sid * _RPW

        @pl.when(gather)
        def _():
            pltpu.sync_copy(src_hbm.at[pl.ds(base_row, _KGRP)], isrc)
            for j in range(2):
                pltpu.async_copy(g_hbm.at[isrc.at[j]], bufs[j], gsem[j])

        _zero_acc(zbuf, acc, sid)

        @pl.when(ones)
        def _():
            _fill_rows(bufs[0], 1.0)
            _fill_rows(bufs[1], 1.0)

        plsc.subcore_barrier()

        @pl.loop(0, _RPW // _KGRP)
        def _(grp):
            r0 = base_row + grp * _KGRP
            pltpu.sync_copy(dst_hbm.at[pl.ds(r0, _KGRP)], idst)

            @pl.when(jnp.logical_and(gather, grp > 0))
            def _():
                pltpu.sync_copy(src_hbm.at[pl.ds(r0, _KGRP)], isrc)
                for j in range(2):
                    pltpu.async_copy(g_hbm.at[isrc.at[j]], bufs[j], gsem[j])

            for j in range(_KGRP):
                b = j % 2

                @pl.when(gather)
                def _():
                    pltpu.make_async_copy(g_hbm.at[isrc.at[j]], bufs[b],
                                          gsem[b]).wait()

                pltpu.async_copy(bufs[b], acc.at[idst.at[j]], ssem[b],
                                 add=True)
                if j + 2 < _KGRP:
                    pltpu.make_async_copy(bufs[b], acc.at[idst.at[j]],
                                          ssem[b]).wait()

                    @pl.when(gather)
                    def _():
                        pltpu.async_copy(g_hbm.at[isrc.at[j + 2]], bufs[b],
                                         gsem[b])

            for j in (_KGRP - 2, _KGRP - 1):
                pltpu.make_async_copy(bufs[j % 2], acc.at[idst.at[j]],
                                      ssem[j % 2]).wait()

        plsc.subcore_barrier()
        _drain_acc(acc, out_hbm, cid, sid)

    return k(g, src2d, dst2d, ones_mode)


def _tc_head(xp, w_init_p, b_init, d0, d1, w0):
    def body(x_ref, wi_ref, bi_ref, d0_ref, d1_ref, w0_ref,
             ih_ref, g0_ref, dinv_ref):
        deg = d0_ref[...] + d1_ref[...] + 1.0
        dinv = lax.rsqrt(deg)
        dinv_ref[...] = dinv
        ih = jnp.dot(x_ref[...], wi_ref[...],
                     preferred_element_type=jnp.float32) + bi_ref[...]
        ih_ref[...] = ih
        y = jnp.dot(ih, w0_ref[...], preferred_element_type=jnp.float32)
        g0_ref[...] = y * dinv

    return pl.pallas_call(
        body,
        out_shape=[
            jax.ShapeDtypeStruct((_N, _D), jnp.float32),
            jax.ShapeDtypeStruct((_N, _D), jnp.float32),
            jax.ShapeDtypeStruct((_N, 1), jnp.float32),
        ],
    )(xp, w_init_p, b_init, d0, d1, w0)


def _tc_mid(p0, p1, g_prev, dinv, b, w_next):
    def body(p0_ref, p1_ref, g_ref, dinv_ref, b_ref, w_ref, out_ref):
        dinv = dinv_ref[...]
        z = dinv * (p0_ref[...] + p1_ref[...] + g_ref[...]) + b_ref[...]
        h = jnp.maximum(z, 0.0)
        out_ref[...] = dinv * jnp.dot(h, w_ref[...],
                                      preferred_element_type=jnp.float32)

    return pl.pallas_call(
        body,
        out_shape=jax.ShapeDtypeStruct((_N, _D), jnp.float32),
    )(p0, p1, g_prev, dinv, b, w_next)


def _tc_tail(p0, p1, g_prev, dinv, b, init_h):
    def body(p0_ref, p1_ref, g_ref, dinv_ref, b_ref, ih_ref, out_ref):
        z = dinv_ref[...] * (p0_ref[...] + p1_ref[...] + g_ref[...]) + b_ref[...]
        out_ref[...] = z + ih_ref[...]

    return pl.pallas_call(
        body,
        out_shape=jax.ShapeDtypeStruct((_N, _D), jnp.float32),
    )(p0, p1, g_prev, dinv, b, init_h)


def kernel(x, edge_index, W_init, b_init, W0, b0, W1, b1, W2, b2):
    src = edge_index[0].astype(jnp.int32)
    dst = edge_index[1].astype(jnp.int32)
    pad = _EPAD - _E
    src2d = jnp.concatenate([src, jnp.zeros((pad,), jnp.int32)]).reshape(_RTOT, _LPR)
    dst2d = jnp.concatenate([dst, jnp.full((pad,), _N, jnp.int32)]).reshape(_RTOT, _LPR)

    xp = jnp.pad(x, ((0, 0), (0, 8 - x.shape[1])))
    w_init_p = jnp.pad(W_init, ((0, 8 - W_init.shape[0]), (0, 0)))
    bi = b_init.reshape(1, _D)
    b0r = b0.reshape(1, _D)
    b1r = b1.reshape(1, _D)
    b2r = b2.reshape(1, _D)

    mode_hist = jnp.ones((8, _LPR), jnp.int32)
    mode_edge = jnp.zeros((8, _LPR), jnp.int32)

    dparts = _sc_segment_sum(jnp.zeros((_N, _D), jnp.float32), src2d, dst2d,
                             mode_hist)
    d0 = dparts[0, :_N, :1]
    d1 = dparts[1, :_N, :1]

    init_h, g0, dinv = _tc_head(xp, w_init_p, bi, d0, d1, W0)

    p = _sc_segment_sum(g0, src2d, dst2d, mode_edge)
    g1 = _tc_mid(p[0, :_N], p[1, :_N], g0, dinv, b0r, W1)

    p = _sc_segment_sum(g1, src2d, dst2d, mode_edge)
    g2 = _tc_mid(p[0, :_N], p[1, :_N], g1, dinv, b1r, W2)

    p = _sc_segment_sum(g2, src2d, dst2d, mode_edge)
    out = _tc_tail(p[0, :_N], p[1, :_N], g2, dinv, b2r, init_h)

    return (out, init_h)
